# Initial kernel scaffold; baseline (speedup 1.0000x reference)
import jax
import jax.numpy as jnp
from jax.experimental import pallas as pl


def kernel(vocab_id_list, table):
    raise NotImplementedError("write your pallas kernel here")


# speedup vs baseline: 2.6930x; 2.6930x over previous
import functools

import jax
import jax.numpy as jnp
from jax import lax
from jax.experimental import pallas as pl
from jax.experimental.pallas import tpu as pltpu
from jax.experimental.pallas import tpu_sc as plsc

_GROUP = 128
_NUM_WORKERS = 32


def kernel(vocab_id_list, table):
    B, L = vocab_id_list.shape
    V, D = table.shape
    N = B * L
    n_groups = N // _GROUP
    g_per_w = n_groups // _NUM_WORKERS

    idx2d = vocab_id_list.reshape(n_groups, _GROUP)

    mesh = plsc.VectorSubcoreMesh(core_axis_name="c", subcore_axis_name="s")

    @functools.partial(
        pl.kernel,
        out_type=jax.ShapeDtypeStruct((N, D), jnp.float32),
        mesh=mesh,
        compiler_params=pltpu.CompilerParams(use_tc_tiling_on_sc=False),
        scratch_types=[
            pltpu.VMEM((g_per_w, _GROUP), jnp.int32),
            pltpu.VMEM((_GROUP, D), jnp.float32),
            pltpu.SemaphoreType.DMA,
        ],
    )
    def sc_gather(idx_hbm, table_hbm, out_hbm, idx_v, rows_v, sem):
        wid = lax.axis_index("s") * 2 + lax.axis_index("c")
        gbase = wid * g_per_w
        pltpu.sync_copy(idx_hbm.at[pl.ds(gbase, g_per_w)], idx_v)

        @pl.loop(0, g_per_w)
        def _(j):
            pltpu.async_copy(table_hbm.at[idx_v.at[j]], rows_v, sem).wait()
            pltpu.sync_copy(rows_v, out_hbm.at[pl.ds((gbase + j) * _GROUP, _GROUP)])

    out = sc_gather(idx2d, table)
    return out.reshape(B, L, D)


# speedup vs baseline: 3.0073x; 1.1167x over previous
import functools

import jax
import jax.numpy as jnp
from jax import lax
from jax.experimental import pallas as pl
from jax.experimental.pallas import tpu as pltpu
from jax.experimental.pallas import tpu_sc as plsc

_GROUP = 128
_NUM_WORKERS = 32


def kernel(vocab_id_list, table):
    B, L = vocab_id_list.shape
    V, D = table.shape
    N = B * L
    n_groups = N // _GROUP
    g_per_w = n_groups // _NUM_WORKERS

    K = 10
    CH = K * _GROUP
    n_chunks = g_per_w // K

    idx2d = vocab_id_list.reshape(n_groups, _GROUP)

    mesh = plsc.VectorSubcoreMesh(core_axis_name="c", subcore_axis_name="s")

    @functools.partial(
        pl.kernel,
        out_type=jax.ShapeDtypeStruct((N, D), jnp.float32),
        mesh=mesh,
        compiler_params=pltpu.CompilerParams(use_tc_tiling_on_sc=False),
        scratch_types=[
            pltpu.VMEM((g_per_w, _GROUP), jnp.int32),
            pltpu.VMEM((CH, D), jnp.float32),
            pltpu.VMEM((CH, D), jnp.float32),
            pltpu.SemaphoreType.DMA,
        ],
    )
    def sc_gather(idx_hbm, table_hbm, out_hbm, idx_v, buf0, buf1, gsem):
        wid = lax.axis_index("s") * 2 + lax.axis_index("c")
        gbase = wid * g_per_w
        row_base = gbase * _GROUP
        pltpu.sync_copy(idx_hbm.at[pl.ds(gbase, g_per_w)], idx_v)

        def fire(c, buf):
            for k in range(K):
                pltpu.async_copy(
                    table_hbm.at[idx_v.at[c * K + k]],
                    buf.at[pl.ds(k * _GROUP, _GROUP)],
                    gsem,
                )

        def drain(buf):
            pltpu.make_async_copy(table_hbm.at[pl.ds(0, CH)], buf, gsem).wait()

        def put(c, buf):
            pltpu.sync_copy(buf, out_hbm.at[pl.ds(row_base + c * CH, CH)])

        fire(0, buf0)

        @pl.loop(0, n_chunks, step=2)
        def _(c):
            drain(buf0)

            @pl.when(c + 1 < n_chunks)
            def _():
                fire(c + 1, buf1)

            put(c, buf0)

            @pl.when(c + 1 < n_chunks)
            def _():
                drain(buf1)

                @pl.when(c + 2 < n_chunks)
                def _():
                    fire(c + 2, buf0)

                put(c + 1, buf1)

    out = sc_gather(idx2d, table)
    return out.reshape(B, L, D)
